# Initial kernel scaffold; baseline (speedup 1.0000x reference)
import jax
import jax.numpy as jnp
from jax.experimental import pallas as pl


def kernel(x):
    raise NotImplementedError("write your pallas kernel here")


# speedup vs baseline: 4.6234x; 4.6234x over previous
import functools

import jax
import jax.numpy as jnp
from jax import lax
from jax.experimental import pallas as pl
from jax.experimental.pallas import tpu as pltpu
from jax.experimental.pallas import tpu_sc as plsc

_TOPK = 1024
_B = 128
_N = 32768
_L = 16
_NV = _N // _L
_NBINS = 256
_NW = 32
_ROWS_PER_W = _B // _NW

_INT_MIN = -2147483648


def _splat(s):
    return lax.broadcast_in_dim(jnp.int32(s), (_L,), ())


def _mono_u(xv):
    b = lax.bitcast_convert_type(xv, jnp.int32)
    m = lax.shift_right_arithmetic(b, _splat(31))
    return b ^ (m | _splat(_INT_MIN))


def _body(x_hbm, out_hbm, row_v, h0, h1, h2, h3):
    nc = 2
    wid = lax.axis_index("s") * nc + lax.axis_index("c")
    iota = lax.iota(jnp.int32, _L)
    lane_base = iota * _NBINS
    ones = jnp.full((_L,), 1, jnp.int32)
    zeros = jnp.full((_L,), 0, jnp.int32)
    hists = (h0, h1, h2, h3)

    for r in range(_ROWS_PER_W):
        row = wid * _ROWS_PER_W + r
        base = row * _N
        pltpu.sync_copy(x_hbm.at[pl.ds(base, _N)], row_v)

        def zero_body(i, c):
            off = i * _L
            for h in hists:
                h[pl.ds(off, _L)] = zeros
            return c
        lax.fori_loop(0, _NBINS, zero_body, 0)

        def hist_pass(h, shift, prefix):
            def body(i, c):
                xv = row_v[pl.ds(i * _L, _L)]
                u = _mono_u(xv)
                bin_ = lax.shift_right_logical(u, _splat(shift)) & _splat(0xFF)
                idx = lane_base + bin_
                if prefix is None:
                    plsc.addupdate_scatter(h, [idx], ones)
                else:
                    msk = lax.shift_right_logical(u, _splat(shift + 8)) == prefix
                    plsc.addupdate_scatter(h, [idx], ones, mask=msk)
                return c
            lax.fori_loop(0, _NV, body, 0)

        def level_scan(h, kr):
            def body(c_rev, carry):
                found, bstar, above, run = carry
                c = 15 - c_rev
                off = c * _L
                t = zeros
                for j in range(_L):
                    t = t + h[pl.ds(j * _NBINS + off, _L)]
                rv = lax.rev(t, (0,))
                cs = plsc.cumsum(rv)
                acc = run + cs
                crossed = acc >= kr
                npop = plsc.all_reduce_population_count(crossed)
                any_c = npop > 0
                j1 = plsc.all_reduce_ffs(crossed)
                sel = iota == j1
                a_at = jnp.sum(jnp.where(sel, acc, 0))
                t_at = jnp.sum(jnp.where(sel, rv, 0))
                bin_here = _splat(0) + (c * _L + 15 - j1)
                above_here = zeros + (a_at - t_at)
                take = jnp.logical_and(jnp.logical_not(found), any_c)
                bstar = jnp.where(take, bin_here, bstar)
                above = jnp.where(take, above_here, above)
                found = jnp.logical_or(found, any_c)
                run = run + (zeros + jnp.sum(t))
                return found, bstar, above, run
            init = (iota < 0, zeros, zeros, zeros)
            _, bstar, above, _ = lax.fori_loop(0, 16, body, init)
            return bstar, above

        kr = _splat(_TOPK)
        prefix = None
        for lvl in range(4):
            shift = 24 - 8 * lvl
            hist_pass(hists[lvl], shift, prefix)
            bstar, above = level_scan(hists[lvl], kr)
            kr = kr - above
            prefix = bstar if prefix is None else ((prefix << _splat(8)) | bstar)

        thr = prefix ^ _splat(_INT_MIN)

        def out_body(i, c):
            xv = row_v[pl.ds(i * _L, _L)]
            b = lax.bitcast_convert_type(xv, jnp.int32)
            m = lax.shift_right_arithmetic(b, _splat(31))
            w = b ^ lax.shift_right_logical(m, _splat(1))
            keep = w >= thr
            row_v[pl.ds(i * _L, _L)] = jnp.where(keep, xv, 0.0)
            return c
        lax.fori_loop(0, _NV, out_body, 0)

        pltpu.sync_copy(row_v, out_hbm.at[pl.ds(base, _N)])


@jax.jit
def kernel(x):
    mesh = plsc.VectorSubcoreMesh(core_axis_name="c", subcore_axis_name="s")
    fn = pl.kernel(
        _body,
        out_type=jax.ShapeDtypeStruct((_B * _N,), jnp.float32),
        mesh=mesh,
        compiler_params=pltpu.CompilerParams(needs_layout_passes=False),
        scratch_types=[
            pltpu.VMEM((_N,), jnp.float32),
            pltpu.VMEM((_L * _NBINS,), jnp.int32),
            pltpu.VMEM((_L * _NBINS,), jnp.int32),
            pltpu.VMEM((_L * _NBINS,), jnp.int32),
            pltpu.VMEM((_L * _NBINS,), jnp.int32),
        ],
    )
    return fn(x.reshape(-1)).reshape(x.shape)


# speedup vs baseline: 5.8212x; 1.2591x over previous
import functools

import jax
import jax.numpy as jnp
from jax import lax
from jax.experimental import pallas as pl
from jax.experimental.pallas import tpu as pltpu
from jax.experimental.pallas import tpu_sc as plsc

_TOPK = 1024
_B = 128
_N = 32768
_L = 16
_NV = _N // _L
_NBINS = 256
_NW = 32
_ROWS_PER_W = _B // _NW

_INT_MIN = -2147483648


def _splat(s):
    return lax.broadcast_in_dim(jnp.int32(s), (_L,), ())


def _mono_u(xv):
    b = lax.bitcast_convert_type(xv, jnp.int32)
    m = lax.shift_right_arithmetic(b, _splat(31))
    return b ^ (m | _splat(_INT_MIN))


def _body(x_hbm, out_hbm, row_v, u_v, h0, h1, h2, h3):
    nc = 2
    wid = lax.axis_index("s") * nc + lax.axis_index("c")
    iota = lax.iota(jnp.int32, _L)
    lane_base = iota * _NBINS
    ones = jnp.full((_L,), 1, jnp.int32)
    zeros = jnp.full((_L,), 0, jnp.int32)
    hists = (h0, h1, h2, h3)
    unroll = 8

    for r in range(_ROWS_PER_W):
        row = wid * _ROWS_PER_W + r
        base = row * _N
        pltpu.sync_copy(x_hbm.at[pl.ds(base, _N)], row_v)

        def zero_body(i, c):
            for k in range(4):
                off = (i * 4 + k) * _L
                for h in hists:
                    h[pl.ds(off, _L)] = zeros
            return c
        lax.fori_loop(0, _NBINS // 4, zero_body, 0)

        def l1_body(i, c):
            for k in range(unroll):
                off = (i * unroll + k) * _L
                xv = row_v[pl.ds(off, _L)]
                u = _mono_u(xv)
                u_v[pl.ds(off, _L)] = u
                bin_ = lax.shift_right_logical(u, _splat(24))
                plsc.addupdate_scatter(h0, [lane_base + bin_], ones)
            return c
        lax.fori_loop(0, _NV // unroll, l1_body, 0)

        def hist_pass(h, shift, prefix):
            def body(i, c):
                for k in range(unroll):
                    off = (i * unroll + k) * _L
                    u = u_v[pl.ds(off, _L)]
                    bin_ = lax.shift_right_logical(u, _splat(shift)) & _splat(0xFF)
                    msk = lax.shift_right_logical(u, _splat(shift + 8)) == prefix
                    plsc.addupdate_scatter(h, [lane_base + bin_], ones, mask=msk)
                return c
            lax.fori_loop(0, _NV // unroll, body, 0)

        def level_scan(h, kr):
            def body(c_rev, carry):
                found, bstar, above, run = carry
                c = 15 - c_rev
                off = c * _L
                t = zeros
                for j in range(_L):
                    t = t + h[pl.ds(j * _NBINS + off, _L)]
                rv = lax.rev(t, (0,))
                cs = plsc.cumsum(rv)
                acc = run + cs
                crossed = acc >= kr
                npop = plsc.all_reduce_population_count(crossed)
                any_c = npop > 0
                j1 = plsc.all_reduce_ffs(crossed)
                sel = iota == j1
                a_at = jnp.sum(jnp.where(sel, acc, 0))
                t_at = jnp.sum(jnp.where(sel, rv, 0))
                bin_here = _splat(0) + (c * _L + 15 - j1)
                above_here = zeros + (a_at - t_at)
                take = jnp.logical_and(jnp.logical_not(found), any_c)
                bstar = jnp.where(take, bin_here, bstar)
                above = jnp.where(take, above_here, above)
                found = jnp.logical_or(found, any_c)
                run = run + (zeros + jnp.sum(t))
                return found, bstar, above, run
            init = (iota < 0, zeros, zeros, zeros)
            _, bstar, above, _ = lax.fori_loop(0, 16, body, init)
            return bstar, above

        kr = _splat(_TOPK)
        prefix = None
        for lvl in range(4):
            shift = 24 - 8 * lvl
            if lvl > 0:
                hist_pass(hists[lvl], shift, prefix)
            bstar, above = level_scan(hists[lvl], kr)
            kr = kr - above
            prefix = bstar if prefix is None else ((prefix << _splat(8)) | bstar)

        thr = prefix ^ _splat(_INT_MIN)

        def out_body(i, c):
            for k in range(unroll):
                off = (i * unroll + k) * _L
                xv = row_v[pl.ds(off, _L)]
                w = u_v[pl.ds(off, _L)] ^ _splat(_INT_MIN)
                keep = w >= thr
                row_v[pl.ds(off, _L)] = jnp.where(keep, xv, 0.0)
            return c
        lax.fori_loop(0, _NV // unroll, out_body, 0)

        pltpu.sync_copy(row_v, out_hbm.at[pl.ds(base, _N)])


@jax.jit
def kernel(x):
    mesh = plsc.VectorSubcoreMesh(core_axis_name="c", subcore_axis_name="s")
    fn = pl.kernel(
        _body,
        out_type=jax.ShapeDtypeStruct((_B * _N,), jnp.float32),
        mesh=mesh,
        compiler_params=pltpu.CompilerParams(needs_layout_passes=False),
        scratch_types=[
            pltpu.VMEM((_N,), jnp.float32),
            pltpu.VMEM((_N,), jnp.int32),
            pltpu.VMEM((_L * _NBINS,), jnp.int32),
            pltpu.VMEM((_L * _NBINS,), jnp.int32),
            pltpu.VMEM((_L * _NBINS,), jnp.int32),
            pltpu.VMEM((_L * _NBINS,), jnp.int32),
        ],
    )
    return fn(x.reshape(-1)).reshape(x.shape)


# speedup vs baseline: 14.7762x; 2.5384x over previous
import functools

import jax
import jax.numpy as jnp
from jax import lax
from jax.experimental import pallas as pl
from jax.experimental.pallas import tpu as pltpu
from jax.experimental.pallas import tpu_sc as plsc

_TOPK = 1024
_B = 128
_N = 32768
_L = 16
_NV = _N // _L
_NBINS = 256
_NW = 32
_ROWS_PER_W = _B // _NW

_INT_MIN = -2147483648


def _splat(s):
    return lax.broadcast_in_dim(jnp.int32(s), (_L,), ())


def _mono_u(xv):
    b = lax.bitcast_convert_type(xv, jnp.int32)
    m = lax.shift_right_arithmetic(b, _splat(31))
    return b ^ (m | _splat(_INT_MIN))


def _body(x_hbm, out_hbm, row_v, u_v, h0, h1, h2, h3):
    nc = 2
    wid = lax.axis_index("s") * nc + lax.axis_index("c")
    iota = lax.iota(jnp.int32, _L)
    lane_base = iota * _NBINS
    ones = jnp.full((_L,), 1, jnp.int32)
    zeros = jnp.full((_L,), 0, jnp.int32)
    hists = (h0, h1, h2, h3)
    unroll = 8

    for r in range(_ROWS_PER_W):
        row = wid * _ROWS_PER_W + r
        base = row * _N
        pltpu.sync_copy(x_hbm.at[pl.ds(base, _N)], row_v)

        @plsc.parallel_loop(0, _NBINS, unroll=4)
        def _(i):
            off = i * _L
            for h in hists:
                h[pl.ds(off, _L)] = zeros

        @plsc.parallel_loop(0, _NV, unroll=unroll)
        def _(i):
            off = i * _L
            xv = row_v[pl.ds(off, _L)]
            u = _mono_u(xv)
            u_v[pl.ds(off, _L)] = u
            bin_ = lax.shift_right_logical(u, _splat(24))
            plsc.addupdate_scatter(h0, [lane_base + bin_], ones)

        def hist_pass(h, shift, prefix):
            @plsc.parallel_loop(0, _NV, unroll=unroll)
            def _(i):
                off = i * _L
                u = u_v[pl.ds(off, _L)]
                bin_ = lax.shift_right_logical(u, _splat(shift)) & _splat(0xFF)
                msk = lax.shift_right_logical(u, _splat(shift + 8)) == prefix
                plsc.addupdate_scatter(h, [lane_base + bin_], ones, mask=msk)

        def level_scan(h, kr):
            def body(c_rev, carry):
                found, bstar, above, run = carry
                c = 15 - c_rev
                off = c * _L
                t = zeros
                for j in range(_L):
                    t = t + h[pl.ds(j * _NBINS + off, _L)]
                rv = lax.rev(t, (0,))
                cs = plsc.cumsum(rv)
                acc = run + cs
                crossed = acc >= kr
                npop = plsc.all_reduce_population_count(crossed)
                any_c = npop > 0
                j1 = plsc.all_reduce_ffs(crossed)
                sel = iota == j1
                a_at = jnp.sum(jnp.where(sel, acc, 0))
                t_at = jnp.sum(jnp.where(sel, rv, 0))
                bin_here = _splat(0) + (c * _L + 15 - j1)
                above_here = zeros + (a_at - t_at)
                take = jnp.logical_and(jnp.logical_not(found), any_c)
                bstar = jnp.where(take, bin_here, bstar)
                above = jnp.where(take, above_here, above)
                found = jnp.logical_or(found, any_c)
                run = run + (zeros + jnp.sum(t))
                return found, bstar, above, run
            init = (iota < 0, zeros, zeros, zeros)
            _, bstar, above, _ = lax.fori_loop(0, 16, body, init)
            return bstar, above

        kr = _splat(_TOPK)
        prefix = None
        for lvl in range(4):
            shift = 24 - 8 * lvl
            if lvl > 0:
                hist_pass(hists[lvl], shift, prefix)
            bstar, above = level_scan(hists[lvl], kr)
            kr = kr - above
            prefix = bstar if prefix is None else ((prefix << _splat(8)) | bstar)

        thr = prefix ^ _splat(_INT_MIN)

        @plsc.parallel_loop(0, _NV, unroll=unroll)
        def _(i):
            off = i * _L
            xv = row_v[pl.ds(off, _L)]
            w = u_v[pl.ds(off, _L)] ^ _splat(_INT_MIN)
            keep = w >= thr
            row_v[pl.ds(off, _L)] = jnp.where(keep, xv, 0.0)

        pltpu.sync_copy(row_v, out_hbm.at[pl.ds(base, _N)])


@jax.jit
def kernel(x):
    mesh = plsc.VectorSubcoreMesh(core_axis_name="c", subcore_axis_name="s")
    fn = pl.kernel(
        _body,
        out_type=jax.ShapeDtypeStruct((_B * _N,), jnp.float32),
        mesh=mesh,
        compiler_params=pltpu.CompilerParams(
            needs_layout_passes=False, disable_bounds_checks=True),
        scratch_types=[
            pltpu.VMEM((_N,), jnp.float32),
            pltpu.VMEM((_N,), jnp.int32),
            pltpu.VMEM((_L * _NBINS,), jnp.int32),
            pltpu.VMEM((_L * _NBINS,), jnp.int32),
            pltpu.VMEM((_L * _NBINS,), jnp.int32),
            pltpu.VMEM((_L * _NBINS,), jnp.int32),
        ],
    )
    return fn(x.reshape(-1)).reshape(x.shape)


# speedup vs baseline: 15.7421x; 1.0654x over previous
import jax
import jax.numpy as jnp
from jax import lax
from jax.experimental import pallas as pl
from jax.experimental.pallas import tpu as pltpu
from jax.experimental.pallas import tpu_sc as plsc

_TOPK = 1024
_B = 128
_N = 32768
_L = 16
_NV = _N // _L
_NBINS = 256
_NW = 32
_ROWS_PER_W = _B // _NW

_INT_MIN = -2147483648


def _splat(s):
    return lax.broadcast_in_dim(jnp.int32(s), (_L,), ())


def _mono_u(xv):
    b = lax.bitcast_convert_type(xv, jnp.int32)
    m = lax.shift_right_arithmetic(b, _splat(31))
    return b ^ (m | _splat(_INT_MIN))


def _body(x_hbm, out_hbm, a0, a1, u_v, h0, h1, h2, in_sem, out_sem):
    nc = 2
    wid = lax.axis_index("s") * nc + lax.axis_index("c")
    iota = lax.iota(jnp.int32, _L)
    lane_base = iota * _NBINS
    ones = jnp.full((_L,), 1, jnp.int32)
    zeros = jnp.full((_L,), 0, jnp.int32)
    bufs = (a0, a1)
    unroll = 8

    def in_copy(r):
        base = (wid * _ROWS_PER_W + r) * _N
        return pltpu.async_copy(x_hbm.at[pl.ds(base, _N)], bufs[r % 2], in_sem)

    def out_copy(r):
        base = (wid * _ROWS_PER_W + r) * _N
        return pltpu.async_copy(bufs[r % 2], out_hbm.at[pl.ds(base, _N)], out_sem)

    pend_out = [None] * _ROWS_PER_W
    h_in = in_copy(0)

    for r in range(_ROWS_PER_W):
        a = bufs[r % 2]

        @plsc.parallel_loop(0, _NBINS, unroll=4)
        def _(i):
            off = i * _L
            for h in (h0, h1, h2):
                h[pl.ds(off, _L)] = zeros

        h_in.wait()

        @plsc.parallel_loop(0, _NV, unroll=unroll)
        def _(i):
            off = i * _L
            xv = a[pl.ds(off, _L)]
            u = _mono_u(xv)
            u_v[pl.ds(off, _L)] = u
            bin_ = lax.shift_right_logical(u, _splat(24))
            plsc.addupdate_scatter(h0, [lane_base + bin_], ones)

        if r + 1 < _ROWS_PER_W:
            if r - 1 >= 0:
                pend_out[r - 1].wait()
            h_in = in_copy(r + 1)

        def hist_pass(h, shift, prefix):
            @plsc.parallel_loop(0, _NV, unroll=unroll)
            def _(i):
                off = i * _L
                u = u_v[pl.ds(off, _L)]
                bin_ = lax.shift_right_logical(u, _splat(shift)) & _splat(0xFF)
                msk = lax.shift_right_logical(u, _splat(shift + 8)) == prefix
                plsc.addupdate_scatter(h, [lane_base + bin_], ones, mask=msk)

        def level_scan(h, kr):
            def body(c_rev, carry):
                found, bstar, above, run = carry
                c = 15 - c_rev
                off = c * _L
                t = zeros
                for j in range(_L):
                    t = t + h[pl.ds(j * _NBINS + off, _L)]
                rv = lax.rev(t, (0,))
                cs = plsc.cumsum(rv)
                acc = run + cs
                crossed = acc >= kr
                npop = plsc.all_reduce_population_count(crossed)
                any_c = npop > 0
                j1 = plsc.all_reduce_ffs(crossed)
                sel = iota == j1
                a_at = jnp.sum(jnp.where(sel, acc, 0))
                t_at = jnp.sum(jnp.where(sel, rv, 0))
                bin_here = _splat(0) + (c * _L + 15 - j1)
                above_here = zeros + (a_at - t_at)
                take = jnp.logical_and(jnp.logical_not(found), any_c)
                bstar = jnp.where(take, bin_here, bstar)
                above = jnp.where(take, above_here, above)
                found = jnp.logical_or(found, any_c)
                run = run + (zeros + jnp.sum(t))
                return found, bstar, above, run
            init = (iota < 0, zeros, zeros, zeros)
            _, bstar, above, _ = lax.fori_loop(0, 16, body, init)
            return bstar, above

        kr = _splat(_TOPK)
        prefix = None
        hist_for_lvl = (h0, h1, h2, h0)
        for lvl in range(4):
            shift = 24 - 8 * lvl
            if lvl == 3:
                @plsc.parallel_loop(0, _NBINS, unroll=4)
                def _(i):
                    h0[pl.ds(i * _L, _L)] = zeros
            if lvl > 0:
                hist_pass(hist_for_lvl[lvl], shift, prefix)
            bstar, above = level_scan(hist_for_lvl[lvl], kr)
            kr = kr - above
            prefix = bstar if prefix is None else ((prefix << _splat(8)) | bstar)

        thr = prefix ^ _splat(_INT_MIN)

        @plsc.parallel_loop(0, _NV, unroll=unroll)
        def _(i):
            off = i * _L
            w = u_v[pl.ds(off, _L)] ^ _splat(_INT_MIN)
            keep = w >= thr
            m2 = lax.shift_right_arithmetic(w, _splat(31))
            b = w ^ lax.shift_right_logical(m2, _splat(1))
            xv = lax.bitcast_convert_type(b, jnp.float32)
            a[pl.ds(off, _L)] = jnp.where(keep, xv, 0.0)

        pend_out[r] = out_copy(r)

    pend_out[_ROWS_PER_W - 2].wait()
    pend_out[_ROWS_PER_W - 1].wait()


@jax.jit
def kernel(x):
    mesh = plsc.VectorSubcoreMesh(core_axis_name="c", subcore_axis_name="s")
    fn = pl.kernel(
        _body,
        out_type=jax.ShapeDtypeStruct((_B * _N,), jnp.float32),
        mesh=mesh,
        compiler_params=pltpu.CompilerParams(
            needs_layout_passes=False, disable_bounds_checks=True),
        scratch_types=[
            pltpu.VMEM((_N,), jnp.float32),
            pltpu.VMEM((_N,), jnp.float32),
            pltpu.VMEM((_N,), jnp.int32),
            pltpu.VMEM((_L * _NBINS,), jnp.int32),
            pltpu.VMEM((_L * _NBINS,), jnp.int32),
            pltpu.VMEM((_L * _NBINS,), jnp.int32),
            pltpu.SemaphoreType.DMA,
            pltpu.SemaphoreType.DMA,
        ],
    )
    return fn(x.reshape(-1)).reshape(x.shape)


# speedup vs baseline: 20.4465x; 1.2988x over previous
import jax
import jax.numpy as jnp
from jax import lax
from jax.experimental import pallas as pl
from jax.experimental.pallas import tpu as pltpu
from jax.experimental.pallas import tpu_sc as plsc

_TOPK = 1024
_B = 128
_N = 32768
_L = 16
_NBINS = 256
_NW = 32
_ROWS_PER_W = _B // _NW
_NR = 256
_NK = 128 // _L

_INT_MIN = -2147483648


def _splat(s):
    return lax.broadcast_in_dim(jnp.int32(s), (_L,), ())


def _mono_u(xv):
    b = lax.bitcast_convert_type(xv, jnp.int32)
    m = lax.shift_right_arithmetic(b, _splat(31))
    return b ^ (m | _splat(_INT_MIN))


def _body(x_hbm, out_hbm, a0, a1, u_v, h0, h1, h2, in_sem, out_sem):
    nc = 2
    wid = lax.axis_index("s") * nc + lax.axis_index("c")
    iota = lax.iota(jnp.int32, _L)
    lane_base = iota * _NBINS
    ones = jnp.full((_L,), 1, jnp.int32)
    zeros = jnp.full((_L,), 0, jnp.int32)
    bufs = (a0, a1)
    unroll = 2

    def in_copy(r):
        row = wid * _ROWS_PER_W + r
        return pltpu.async_copy(
            x_hbm.at[row // 8, :, row % 8, :], bufs[r % 2], in_sem)

    def out_copy(r):
        row = wid * _ROWS_PER_W + r
        return pltpu.async_copy(
            bufs[r % 2], out_hbm.at[row // 8, :, row % 8, :], out_sem)

    pend_out = [None] * _ROWS_PER_W
    h_in = in_copy(0)

    for r in range(_ROWS_PER_W):
        a = bufs[r % 2]

        @plsc.parallel_loop(0, _NBINS, unroll=4)
        def _(i):
            off = i * _L
            for h in (h0, h1, h2):
                h[pl.ds(off, _L)] = zeros

        h_in.wait()

        @plsc.parallel_loop(0, _NR, unroll=unroll)
        def _(i):
            for k in range(_NK):
                sl = pl.ds(k * _L, _L)
                u = _mono_u(a[i, sl])
                u_v[i, sl] = u
                bin_ = lax.shift_right_logical(u, _splat(24))
                plsc.addupdate_scatter(h0, [lane_base + bin_], ones)

        if r + 1 < _ROWS_PER_W:
            if r - 1 >= 0:
                pend_out[r - 1].wait()
            h_in = in_copy(r + 1)

        def hist_pass(h, shift, prefix):
            @plsc.parallel_loop(0, _NR, unroll=unroll)
            def _(i):
                for k in range(_NK):
                    u = u_v[i, pl.ds(k * _L, _L)]
                    bin_ = lax.shift_right_logical(u, _splat(shift)) & _splat(0xFF)
                    msk = lax.shift_right_logical(u, _splat(shift + 8)) == prefix
                    plsc.addupdate_scatter(h, [lane_base + bin_], ones, mask=msk)

        def level_scan(h, kr):
            def body(c_rev, carry):
                found, bstar, above, run = carry
                c = 15 - c_rev
                off = c * _L
                t = zeros
                for j in range(_L):
                    t = t + h[pl.ds(j * _NBINS + off, _L)]
                rv = lax.rev(t, (0,))
                cs = plsc.cumsum(rv)
                acc = run + cs
                crossed = acc >= kr
                npop = plsc.all_reduce_population_count(crossed)
                any_c = npop > 0
                j1 = plsc.all_reduce_ffs(crossed)
                sel = iota == j1
                a_at = jnp.sum(jnp.where(sel, acc, 0))
                t_at = jnp.sum(jnp.where(sel, rv, 0))
                bin_here = _splat(0) + (c * _L + 15 - j1)
                above_here = zeros + (a_at - t_at)
                take = jnp.logical_and(jnp.logical_not(found), any_c)
                bstar = jnp.where(take, bin_here, bstar)
                above = jnp.where(take, above_here, above)
                found = jnp.logical_or(found, any_c)
                run = run + (zeros + jnp.sum(t))
                return found, bstar, above, run
            init = (iota < 0, zeros, zeros, zeros)
            _, bstar, above, _ = lax.fori_loop(0, 16, body, init)
            return bstar, above

        kr = _splat(_TOPK)
        prefix = None
        hist_for_lvl = (h0, h1, h2, h0)
        for lvl in range(4):
            shift = 24 - 8 * lvl
            if lvl == 3:
                @plsc.parallel_loop(0, _NBINS, unroll=4)
                def _(i):
                    h0[pl.ds(i * _L, _L)] = zeros
            if lvl > 0:
                hist_pass(hist_for_lvl[lvl], shift, prefix)
            bstar, above = level_scan(hist_for_lvl[lvl], kr)
            kr = kr - above
            prefix = bstar if prefix is None else ((prefix << _splat(8)) | bstar)

        thr = prefix ^ _splat(_INT_MIN)

        @plsc.parallel_loop(0, _NR, unroll=unroll)
        def _(i):
            for k in range(_NK):
                sl = pl.ds(k * _L, _L)
                w = u_v[i, sl] ^ _splat(_INT_MIN)
                keep = w >= thr
                m2 = lax.shift_right_arithmetic(w, _splat(31))
                b = w ^ lax.shift_right_logical(m2, _splat(1))
                xv = lax.bitcast_convert_type(b, jnp.float32)
                a[i, sl] = jnp.where(keep, xv, 0.0)

        pend_out[r] = out_copy(r)

    pend_out[_ROWS_PER_W - 2].wait()
    pend_out[_ROWS_PER_W - 1].wait()


@jax.jit
def kernel(x):
    mesh = plsc.VectorSubcoreMesh(core_axis_name="c", subcore_axis_name="s")
    fn = pl.kernel(
        _body,
        out_type=jax.ShapeDtypeStruct((_B // 8, _NR, 8, 128), jnp.float32),
        mesh=mesh,
        compiler_params=pltpu.CompilerParams(
            needs_layout_passes=False, disable_bounds_checks=True),
        scratch_types=[
            pltpu.VMEM((_NR, 128), jnp.float32),
            pltpu.VMEM((_NR, 128), jnp.float32),
            pltpu.VMEM((_NR, 128), jnp.int32),
            pltpu.VMEM((_L * _NBINS,), jnp.int32),
            pltpu.VMEM((_L * _NBINS,), jnp.int32),
            pltpu.VMEM((_L * _NBINS,), jnp.int32),
            pltpu.SemaphoreType.DMA,
            pltpu.SemaphoreType.DMA,
        ],
    )
    xt = x.reshape(_B // 8, 8, _NR, 128).transpose(0, 2, 1, 3)
    out = fn(xt)
    return out.transpose(0, 2, 1, 3).reshape(_B, _N)


# speedup vs baseline: 20.7616x; 1.0154x over previous
import jax
import jax.numpy as jnp
from jax import lax
from jax.experimental import pallas as pl
from jax.experimental.pallas import tpu as pltpu
from jax.experimental.pallas import tpu_sc as plsc

_TOPK = 1024
_B = 128
_N = 32768
_L = 16
_NBINS = 256
_NW = 32
_ROWS_PER_W = _B // _NW
_NR = 256
_NK = 128 // _L

_INT_MIN = -2147483648


def _splat(s):
    return lax.broadcast_in_dim(jnp.int32(s), (_L,), ())


def _mono_u(xv):
    b = lax.bitcast_convert_type(xv, jnp.int32)
    m = lax.shift_right_arithmetic(b, _splat(31))
    return b ^ (m | _splat(_INT_MIN))


def _body(x_hbm, out_hbm, a0, a1, u_v, h0, h1, h2, in_sem, out_sem):
    nc = 2
    wid = lax.axis_index("s") * nc + lax.axis_index("c")
    iota = lax.iota(jnp.int32, _L)
    lane_base = iota * _NBINS
    ones = jnp.full((_L,), 1, jnp.int32)
    zeros = jnp.full((_L,), 0, jnp.int32)
    bufs = (a0, a1)
    unroll = 1

    def in_copy(r):
        row = wid * _ROWS_PER_W + r
        return pltpu.async_copy(
            x_hbm.at[row // 8, :, row % 8, :], bufs[r % 2], in_sem)

    def out_copy(r):
        row = wid * _ROWS_PER_W + r
        return pltpu.async_copy(
            bufs[r % 2], out_hbm.at[row // 8, :, row % 8, :], out_sem)

    pend_out = [None] * _ROWS_PER_W
    h_in = in_copy(0)

    for r in range(_ROWS_PER_W):
        a = bufs[r % 2]

        @plsc.parallel_loop(0, _NBINS, unroll=4)
        def _(i):
            off = i * _L
            for h in (h0, h1, h2):
                h[pl.ds(off, _L)] = zeros

        h_in.wait()

        @plsc.parallel_loop(0, _NR, unroll=unroll)
        def _(i):
            for k in range(_NK):
                sl = pl.ds(k * _L, _L)
                u = _mono_u(a[i, sl])
                u_v[i, sl] = u
                bin_ = lax.shift_right_logical(u, _splat(24))
                plsc.addupdate_scatter(h0, [lane_base + bin_], ones)

        if r + 1 < _ROWS_PER_W:
            if r - 1 >= 0:
                pend_out[r - 1].wait()
            h_in = in_copy(r + 1)

        def hist_pass(h, shift, prefix):
            @plsc.parallel_loop(0, _NR, unroll=unroll)
            def _(i):
                for k in range(_NK):
                    u = u_v[i, pl.ds(k * _L, _L)]
                    bin_ = lax.shift_right_logical(u, _splat(shift)) & _splat(0xFF)
                    msk = lax.shift_right_logical(u, _splat(shift + 8)) == prefix
                    plsc.addupdate_scatter(h, [lane_base + bin_], ones, mask=msk)

        def level_scan(h, kr):
            def body(c_rev, carry):
                found, bstar, above, run = carry
                c = 15 - c_rev
                off = c * _L
                t = zeros
                for j in range(_L):
                    t = t + h[pl.ds(j * _NBINS + off, _L)]
                rv = lax.rev(t, (0,))
                cs = plsc.cumsum(rv)
                acc = run + cs
                crossed = acc >= kr
                npop = plsc.all_reduce_population_count(crossed)
                any_c = npop > 0
                j1 = plsc.all_reduce_ffs(crossed)
                sel = iota == j1
                a_at = jnp.sum(jnp.where(sel, acc, 0))
                t_at = jnp.sum(jnp.where(sel, rv, 0))
                bin_here = _splat(0) + (c * _L + 15 - j1)
                above_here = zeros + (a_at - t_at)
                take = jnp.logical_and(jnp.logical_not(found), any_c)
                bstar = jnp.where(take, bin_here, bstar)
                above = jnp.where(take, above_here, above)
                found = jnp.logical_or(found, any_c)
                run = run + (zeros + jnp.sum(t))
                return found, bstar, above, run
            init = (iota < 0, zeros, zeros, zeros)
            _, bstar, above, _ = lax.fori_loop(0, 16, body, init)
            return bstar, above

        kr = _splat(_TOPK)
        prefix = None
        hist_for_lvl = (h0, h1, h2, h0)
        for lvl in range(4):
            shift = 24 - 8 * lvl
            if lvl == 3:
                @plsc.parallel_loop(0, _NBINS, unroll=4)
                def _(i):
                    h0[pl.ds(i * _L, _L)] = zeros
            if lvl > 0:
                hist_pass(hist_for_lvl[lvl], shift, prefix)
            bstar, above = level_scan(hist_for_lvl[lvl], kr)
            kr = kr - above
            prefix = bstar if prefix is None else ((prefix << _splat(8)) | bstar)

        thr = prefix ^ _splat(_INT_MIN)

        @plsc.parallel_loop(0, _NR, unroll=unroll)
        def _(i):
            for k in range(_NK):
                sl = pl.ds(k * _L, _L)
                w = u_v[i, sl] ^ _splat(_INT_MIN)
                keep = w >= thr
                m2 = lax.shift_right_arithmetic(w, _splat(31))
                b = w ^ lax.shift_right_logical(m2, _splat(1))
                xv = lax.bitcast_convert_type(b, jnp.float32)
                a[i, sl] = jnp.where(keep, xv, 0.0)

        pend_out[r] = out_copy(r)

    pend_out[_ROWS_PER_W - 2].wait()
    pend_out[_ROWS_PER_W - 1].wait()


@jax.jit
def kernel(x):
    mesh = plsc.VectorSubcoreMesh(core_axis_name="c", subcore_axis_name="s")
    fn = pl.kernel(
        _body,
        out_type=jax.ShapeDtypeStruct((_B // 8, _NR, 8, 128), jnp.float32),
        mesh=mesh,
        compiler_params=pltpu.CompilerParams(
            needs_layout_passes=False, disable_bounds_checks=True),
        scratch_types=[
            pltpu.VMEM((_NR, 128), jnp.float32),
            pltpu.VMEM((_NR, 128), jnp.float32),
            pltpu.VMEM((_NR, 128), jnp.int32),
            pltpu.VMEM((_L * _NBINS,), jnp.int32),
            pltpu.VMEM((_L * _NBINS,), jnp.int32),
            pltpu.VMEM((_L * _NBINS,), jnp.int32),
            pltpu.SemaphoreType.DMA,
            pltpu.SemaphoreType.DMA,
        ],
    )
    xt = x.reshape(_B // 8, 8, _NR, 128).transpose(0, 2, 1, 3)
    out = fn(xt)
    return out.transpose(0, 2, 1, 3).reshape(_B, _N)
